# baseline (speedup 1.0000x reference)
import jax
import jax.numpy as jnp
from jax import lax
from jax.experimental import pallas as pl
from jax.experimental.pallas import tpu as pltpu
from jax.experimental.pallas import tpu_sc as plsc

B, N, L = 4096, 50, 5
D = 64
G = B * N
NC, NS = 2, 16
NW = NC * NS
BC = 2
CG = BC * N
BW = B // NW
NCH = BW // BC
CGF = CG + 4


def _sc_body(uidx, iidx, ust, uvt, ist, ivt,
             us_out, uv_out, is_out, iv_out,
             idx_u, idx_i, rows_u, rows_i, srows_u, srows_i,
             acc_u, acc_i, sacc_u, sacc_i,
             gsem_u, gsem_i, isem_u, isem_i, wbsem_u, wbsem_i):
    wid = lax.axis_index("s") * NC + lax.axis_index("c")
    gbase = wid * BW * N
    bbase = wid * BW

    FU = (uidx, ust, uvt, us_out, uv_out, idx_u, rows_u, srows_u,
          acc_u, sacc_u, gsem_u, isem_u, wbsem_u)
    FI = (iidx, ist, ivt, is_out, iv_out, idx_i, rows_i, srows_i,
          acc_i, sacc_i, gsem_i, isem_i, wbsem_i)

    def fstart(c):
        g0 = gbase + c * CG
        return pl.multiple_of(g0 - (g0 & 7), 8)

    def doff(c):
        g0 = gbase + c * CG
        return g0 & 7

    def fire_idx(f, c):
        idx_hbm, idx_t, isem = f[0], f[5], f[11]
        pltpu.async_copy(idx_hbm.at[:, pl.ds(fstart(c), CGF)], idx_t, isem)

    def wait_idx(f):
        idx_hbm, idx_t, isem = f[0], f[5], f[11]
        pltpu.make_async_copy(idx_hbm.at[:, pl.ds(0, CGF)], idx_t,
                              isem).wait()

    def fire_gathers(f, c):
        st, vt, idx_t, rows_v, srows_v, gsem = (
            f[1], f[2], f[5], f[6], f[7], f[10])
        for l in range(L):
            sel = idx_t.at[l]
            pltpu.async_copy(vt.at[sel], rows_v.at[l], gsem)
            pltpu.async_copy(st.at[sel], srows_v.at[l], gsem)

    def wait_gathers(f):
        st, vt, rows_v, srows_v, gsem = f[1], f[2], f[6], f[7], f[10]
        for l in range(L):
            pltpu.make_async_copy(vt.at[pl.ds(0, CGF)], rows_v.at[l],
                                  gsem).wait()
            pltpu.make_async_copy(st.at[pl.ds(0, CGF)], srows_v.at[l],
                                  gsem).wait()

    def reduce(f, c):
        rows_v, srows_v, acc_v, sacc_v = f[6], f[7], f[8], f[9]
        d = doff(c)

        def vred(n, c2):
            for b in range(BC):
                g = d + b * N + n
                for q in range(D // 16):
                    acc = rows_v[0, g, pl.ds(q * 16, 16)]
                    for l in range(1, L):
                        acc = acc + rows_v[l, g, pl.ds(q * 16, 16)]
                    acc_v[b, n, 0, pl.ds(q * 16, 16)] = acc
            return c2

        lax.fori_loop(0, N, vred, 0, unroll=2)

        for b in range(BC):
            for o in (0, 16, N - 16):
                acc = srows_v[0, pl.ds(d + b * N + o, 16)]
                for l in range(1, L):
                    acc = acc + srows_v[l, pl.ds(d + b * N + o, 16)]
                sacc_v[b, pl.ds(o, 16)] = acc

    def fire_wb(f, c):
        outs, outv, acc_v, sacc_v, wbsem = f[3], f[4], f[8], f[9], f[12]
        b0 = bbase + c * BC
        pltpu.async_copy(acc_v, outv.at[pl.ds(b0, BC)], wbsem)
        pltpu.async_copy(sacc_v, outs.at[pl.ds(b0, BC)], wbsem)

    def wait_wb(f):
        outs, outv, acc_v, sacc_v, wbsem = f[3], f[4], f[8], f[9], f[12]
        pltpu.make_async_copy(acc_v, outv.at[pl.ds(0, BC)], wbsem).wait()
        pltpu.make_async_copy(sacc_v, outs.at[pl.ds(0, BC)], wbsem).wait()

    fire_idx(FI, 0)
    pltpu.sync_copy(uidx.at[:, pl.ds(fstart(0), CGF)], idx_u)
    fire_gathers(FU, 0)

    def chunk_body(c, carry):
        wait_gathers(FU)

        @pl.when(c < NCH - 1)
        def _():
            fire_idx(FU, c + 1)

        wait_idx(FI)

        @pl.when(c > 0)
        def _():
            wait_wb(FI)

        fire_gathers(FI, c)

        @pl.when(c > 0)
        def _():
            wait_wb(FU)

        reduce(FU, c)
        fire_wb(FU, c)

        wait_gathers(FI)

        @pl.when(c < NCH - 1)
        def _():
            fire_idx(FI, c + 1)
            wait_idx(FU)
            fire_gathers(FU, c + 1)

        reduce(FI, c)
        fire_wb(FI, c)
        return carry

    lax.fori_loop(0, NCH, chunk_body, 0)
    wait_wb(FU)
    wait_wb(FI)


def kernel(user_id, item_id, user_scalar_table, user_vector_table,
           item_scalar_table, item_vector_table):
    uidx = user_id.reshape(G, L).T.astype(jnp.int32)
    iidx = item_id.reshape(G, L).T.astype(jnp.int32)

    call = pl.kernel(
        _sc_body,
        out_type=(
            jax.ShapeDtypeStruct((B, N), jnp.float32),
            jax.ShapeDtypeStruct((B, N, 1, D), jnp.float32),
            jax.ShapeDtypeStruct((B, N), jnp.float32),
            jax.ShapeDtypeStruct((B, N, 1, D), jnp.float32),
        ),
        mesh=plsc.VectorSubcoreMesh(core_axis_name="c", subcore_axis_name="s"),
        scratch_types=[
            pltpu.VMEM((L, CGF), jnp.int32),
            pltpu.VMEM((L, CGF), jnp.int32),
            pltpu.VMEM((L, CGF, D), jnp.float32),
            pltpu.VMEM((L, CGF, D), jnp.float32),
            pltpu.VMEM((L, CGF), jnp.float32),
            pltpu.VMEM((L, CGF), jnp.float32),
            pltpu.VMEM((BC, N, 1, D), jnp.float32),
            pltpu.VMEM((BC, N, 1, D), jnp.float32),
            pltpu.VMEM((BC, N), jnp.float32),
            pltpu.VMEM((BC, N), jnp.float32),
            pltpu.SemaphoreType.DMA,
            pltpu.SemaphoreType.DMA,
            pltpu.SemaphoreType.DMA,
            pltpu.SemaphoreType.DMA,
            pltpu.SemaphoreType.DMA,
            pltpu.SemaphoreType.DMA,
        ],
        compiler_params=pltpu.CompilerParams(use_tc_tiling_on_sc=False),
    )
    us, uv, is_, iv = call(
        uidx, iidx,
        user_scalar_table.reshape(-1), user_vector_table,
        item_scalar_table.reshape(-1), item_vector_table)
    return (us.reshape(B, N, 1), uv, is_.reshape(B, N, 1), iv)


# speedup vs baseline: 1.7548x; 1.7548x over previous
import jax
import jax.numpy as jnp
from jax import lax
from jax.experimental import pallas as pl
from jax.experimental.pallas import tpu as pltpu
from jax.experimental.pallas import tpu_sc as plsc

B, N, L = 4096, 50, 5
D = 64
G = B * N
NC, NS = 2, 16
NW = NC * NS
GW = G // NW
CG = 128
NCH = GW // CG


def _sc_body(uidx, iidx, ust, uvt, ist, ivt,
             us_out, uv_out, is_out, iv_out,
             idx_u, idx_i, rows_u, rows_i, srows_u, srows_i,
             acc_u, acc_i, sacc_u, sacc_i,
             gsem_u, gsem_i, isem_u, isem_i, wbsem_u, wbsem_i):
    wid = lax.axis_index("s") * NC + lax.axis_index("c")
    base = wid * GW

    FU = (uidx, ust, uvt, us_out, uv_out, idx_u, rows_u, srows_u,
          acc_u, sacc_u, gsem_u, isem_u, wbsem_u)
    FI = (iidx, ist, ivt, is_out, iv_out, idx_i, rows_i, srows_i,
          acc_i, sacc_i, gsem_i, isem_i, wbsem_i)

    def fire_idx(f, c):
        idx_hbm, _, _, _, _, idx_v, _, _, _, _, _, isem, _ = f
        pltpu.async_copy(idx_hbm.at[:, pl.ds(base + c * CG, CG)], idx_v, isem)

    def wait_idx(f):
        idx_hbm, _, _, _, _, idx_v, _, _, _, _, _, isem, _ = f
        pltpu.make_async_copy(idx_hbm.at[:, pl.ds(0, CG)], idx_v, isem).wait()

    def fire_gathers(f):
        _, st, vt, _, _, idx_v, rows_v, srows_v, _, _, gsem, _, _ = f
        for l in range(L):
            pltpu.async_copy(vt.at[idx_v.at[l]], rows_v.at[l], gsem)
        for l in range(L):
            pltpu.async_copy(st.at[idx_v.at[l]], srows_v.at[l], gsem)

    def wait_gathers(f):
        _, st, vt, _, _, _, rows_v, srows_v, _, _, gsem, _, _ = f
        for l in range(L):
            pltpu.make_async_copy(vt.at[pl.ds(0, CG)], rows_v.at[l],
                                  gsem).wait()
        for l in range(L):
            pltpu.make_async_copy(st.at[pl.ds(0, CG)], srows_v.at[l],
                                  gsem).wait()

    def reduce(f):
        _, _, _, _, _, _, rows_v, srows_v, acc_v, sacc_v, _, _, _ = f

        def vred(g, c2):
            for q in range(D // 16):
                acc = rows_v[0, g, pl.ds(q * 16, 16)]
                for l in range(1, L):
                    acc = acc + rows_v[l, g, pl.ds(q * 16, 16)]
                acc_v[g, pl.ds(q * 16, 16)] = acc
            return c2

        lax.fori_loop(0, CG, vred, 0, unroll=2)

        def sred(t, c2):
            acc = srows_v[0, pl.ds(t * 16, 16)]
            for l in range(1, L):
                acc = acc + srows_v[l, pl.ds(t * 16, 16)]
            sacc_v[pl.ds(t * 16, 16)] = acc
            return c2

        lax.fori_loop(0, CG // 16, sred, 0)

    def fire_wb(f, c):
        _, _, _, outs, outv, _, _, _, acc_v, sacc_v, _, _, wbsem = f
        g0 = base + c * CG
        pltpu.async_copy(acc_v, outv.at[pl.ds(g0, CG)], wbsem)
        pltpu.async_copy(sacc_v, outs.at[pl.ds(g0, CG)], wbsem)

    def wait_wb(f):
        _, _, _, outs, outv, _, _, _, acc_v, sacc_v, _, _, wbsem = f
        pltpu.make_async_copy(acc_v, outv.at[pl.ds(0, CG)], wbsem).wait()
        pltpu.make_async_copy(sacc_v, outs.at[pl.ds(0, CG)], wbsem).wait()

    fire_idx(FI, 0)
    pltpu.sync_copy(uidx.at[:, pl.ds(base, CG)], idx_u)
    fire_gathers(FU)

    def chunk_body(c, carry):
        wait_gathers(FU)

        @pl.when(c < NCH - 1)
        def _():
            fire_idx(FU, c + 1)

        wait_idx(FI)

        @pl.when(c > 0)
        def _():
            wait_wb(FI)

        fire_gathers(FI)

        @pl.when(c > 0)
        def _():
            wait_wb(FU)

        reduce(FU)
        fire_wb(FU, c)

        wait_gathers(FI)

        @pl.when(c < NCH - 1)
        def _():
            fire_idx(FI, c + 1)
            wait_idx(FU)
            fire_gathers(FU)

        reduce(FI)
        fire_wb(FI, c)
        return carry

    lax.fori_loop(0, NCH, chunk_body, 0)
    wait_wb(FU)
    wait_wb(FI)


def kernel(user_id, item_id, user_scalar_table, user_vector_table,
           item_scalar_table, item_vector_table):
    uidx = user_id.reshape(G, L).T.astype(jnp.int32)
    iidx = item_id.reshape(G, L).T.astype(jnp.int32)

    call = pl.kernel(
        _sc_body,
        out_type=(
            jax.ShapeDtypeStruct((G,), jnp.float32),
            jax.ShapeDtypeStruct((G, D), jnp.float32),
            jax.ShapeDtypeStruct((G,), jnp.float32),
            jax.ShapeDtypeStruct((G, D), jnp.float32),
        ),
        mesh=plsc.VectorSubcoreMesh(core_axis_name="c", subcore_axis_name="s"),
        scratch_types=[
            pltpu.VMEM((L, CG), jnp.int32),
            pltpu.VMEM((L, CG), jnp.int32),
            pltpu.VMEM((L, CG, D), jnp.float32),
            pltpu.VMEM((L, CG, D), jnp.float32),
            pltpu.VMEM((L, CG), jnp.float32),
            pltpu.VMEM((L, CG), jnp.float32),
            pltpu.VMEM((CG, D), jnp.float32),
            pltpu.VMEM((CG, D), jnp.float32),
            pltpu.VMEM((CG,), jnp.float32),
            pltpu.VMEM((CG,), jnp.float32),
            pltpu.SemaphoreType.DMA,
            pltpu.SemaphoreType.DMA,
            pltpu.SemaphoreType.DMA,
            pltpu.SemaphoreType.DMA,
            pltpu.SemaphoreType.DMA,
            pltpu.SemaphoreType.DMA,
        ],
        compiler_params=pltpu.CompilerParams(use_tc_tiling_on_sc=False),
    )
    us, uv, is_, iv = call(
        uidx, iidx,
        user_scalar_table.reshape(-1), user_vector_table,
        item_scalar_table.reshape(-1), item_vector_table)
    return (us.reshape(B, N, 1), uv.reshape(B, N, 1, D),
            is_.reshape(B, N, 1), iv.reshape(B, N, 1, D))
